# baseline (speedup 1.0000x reference)
import functools

import jax
import jax.numpy as jnp
from jax import lax
from jax.experimental import pallas as pl
from jax.experimental.pallas import tpu as pltpu
from jax.experimental.pallas import tpu_sc as plsc

D_MODEL = 768
NUM_EXPERTS = 64
TOP_K = 8
BLOCK_R = 2048
SC_CHUNK = 512
N_WORKERS = 32
NEG = -3.0e38


def _matmul_body(x_ref, w_ref, b_ref, logits_ref):
    logits_ref[...] = jnp.dot(
        x_ref[...], w_ref[...], preferred_element_type=jnp.float32
    ) + b_ref[...]


def _tc_logits(x, W, b):
    n_tokens = x.shape[0]
    return pl.pallas_call(
        _matmul_body,
        grid=(n_tokens // BLOCK_R,),
        in_specs=[
            pl.BlockSpec((BLOCK_R, D_MODEL), lambda i: (i, 0)),
            pl.BlockSpec((D_MODEL, NUM_EXPERTS), lambda i: (0, 0)),
            pl.BlockSpec((1, NUM_EXPERTS), lambda i: (0, 0)),
        ],
        out_specs=pl.BlockSpec((BLOCK_R, NUM_EXPERTS), lambda i: (i, 0)),
        out_shape=jax.ShapeDtypeStruct((n_tokens, NUM_EXPERTS), jnp.float32),
        compiler_params=pltpu.CompilerParams(
            dimension_semantics=("arbitrary",),
        ),
    )(x, W, b.reshape(1, NUM_EXPERTS))


def _batch16(in_v, out_v, r0):
    rowoff = lax.iota(jnp.int32, 16) * NUM_EXPERTS + r0 * NUM_EXPERTS

    def col(j):
        return plsc.load_gather(in_v, [rowoff + j])

    s = [jnp.full((16,), NEG, jnp.float32) for _ in range(TOP_K)]
    for j in range(NUM_EXPERTS):
        t = col(j)
        for k in range(TOP_K):
            hi = jnp.maximum(s[k], t)
            t = jnp.minimum(s[k], t)
            s[k] = hi
    t8 = s[TOP_K - 1]
    cnt = jnp.zeros((16,), jnp.int32)
    one = jnp.ones((16,), jnp.int32)
    for j in range(NUM_EXPERTS):
        cnt = cnt + jnp.where(col(j) > t8, one, 0)
    for j in range(NUM_EXPERTS):
        v = col(j)
        gt = v > t8
        eq = v == t8
        sel = gt | (eq & (cnt < TOP_K))
        cnt = cnt + jnp.where(eq, one, 0)
        zz = jnp.minimum(jnp.maximum(v, 0.0) * 2.0, 15.0)
        q = zz.astype(jnp.int32).astype(jnp.float32) * 0.5
        o = jnp.where(sel, q, jnp.float32(0.0))
        plsc.store_scatter(out_v, [rowoff + j], o)


def _sc_route(logits_flat, n_tokens):
    vals_per_worker = (n_tokens // N_WORKERS) * NUM_EXPERTS
    chunk_vals = SC_CHUNK * NUM_EXPERTS
    n_chunks = vals_per_worker // chunk_vals
    mesh = plsc.VectorSubcoreMesh(core_axis_name="c", subcore_axis_name="s")

    @functools.partial(
        pl.kernel, mesh=mesh,
        compiler_params=pltpu.CompilerParams(needs_layout_passes=False),
        out_type=jax.ShapeDtypeStruct((n_tokens * NUM_EXPERTS,), jnp.float32),
        scratch_types=[
            pltpu.VMEM((chunk_vals,), jnp.float32),
            pltpu.VMEM((chunk_vals,), jnp.float32),
        ],
    )
    def route(logits_hbm, out_hbm, in_v, out_v):
        wid = lax.axis_index("s") * 2 + lax.axis_index("c")
        base = wid * vals_per_worker

        def chunk_body(ci, carry):
            cbase = base + ci * chunk_vals
            pltpu.sync_copy(logits_hbm.at[pl.ds(cbase, chunk_vals)], in_v)

            def b_body(bi, c2):
                _batch16(in_v, out_v, bi * 16)
                return c2

            lax.fori_loop(0, SC_CHUNK // 16, b_body, 0)
            pltpu.sync_copy(out_v, out_hbm.at[pl.ds(cbase, chunk_vals)])
            return carry

        lax.fori_loop(0, n_chunks, chunk_body, 0)

    return route(logits_flat)


@jax.jit
def kernel(x, W, b):
    n_tokens = x.shape[0]
    logits = _tc_logits(x, W, b)
    rw = _sc_route(logits.reshape(-1), n_tokens)
    return (rw.reshape(n_tokens, NUM_EXPERTS), logits)


# speedup vs baseline: 1.0687x; 1.0687x over previous
import functools

import jax
import jax.numpy as jnp
from jax import lax
from jax.experimental import pallas as pl
from jax.experimental.pallas import tpu as pltpu
from jax.experimental.pallas import tpu_sc as plsc

D_MODEL = 768
NUM_EXPERTS = 64
TOP_K = 8
BLOCK_R = 2048
SC_CHUNK = 512
N_WORKERS = 32
NEG = -3.0e38


def _matmul_body(x_ref, w_ref, b_ref, logits_ref):
    logits_ref[...] = jnp.dot(
        x_ref[...], w_ref[...], preferred_element_type=jnp.float32
    ) + b_ref[...]


def _tc_logits(x, W, b):
    n_tokens = x.shape[0]
    return pl.pallas_call(
        _matmul_body,
        grid=(n_tokens // BLOCK_R,),
        in_specs=[
            pl.BlockSpec((BLOCK_R, D_MODEL), lambda i: (i, 0)),
            pl.BlockSpec((D_MODEL, NUM_EXPERTS), lambda i: (0, 0)),
            pl.BlockSpec((1, NUM_EXPERTS), lambda i: (0, 0)),
        ],
        out_specs=pl.BlockSpec((BLOCK_R, NUM_EXPERTS), lambda i: (i, 0)),
        out_shape=jax.ShapeDtypeStruct((n_tokens, NUM_EXPERTS), jnp.float32),
        compiler_params=pltpu.CompilerParams(
            dimension_semantics=("arbitrary",),
        ),
    )(x, W, b.reshape(1, NUM_EXPERTS))


def _batch16(in_v, out_v, r0):
    rowoff = lax.iota(jnp.int32, 16) * NUM_EXPERTS + r0 * NUM_EXPERTS

    def col(j):
        return plsc.load_gather(in_v, [rowoff + j])

    sa = [jnp.full((16,), NEG, jnp.float32) for _ in range(TOP_K)]
    sb = [jnp.full((16,), NEG, jnp.float32) for _ in range(TOP_K)]
    half = NUM_EXPERTS // 2
    for j in range(half):
        ta = col(j)
        tb = col(half + j)
        for k in range(TOP_K):
            ha = jnp.maximum(sa[k], ta)
            ta = jnp.minimum(sa[k], ta)
            sa[k] = ha
            hb = jnp.maximum(sb[k], tb)
            tb = jnp.minimum(sb[k], tb)
            sb[k] = hb
    merged = [jnp.maximum(sa[i], sb[TOP_K - 1 - i]) for i in range(TOP_K)]
    t8 = merged[0]
    for i in range(1, TOP_K):
        t8 = jnp.minimum(t8, merged[i])
    one = jnp.ones((16,), jnp.int32)
    m8 = jnp.zeros((16,), jnp.int32)
    for i in range(TOP_K):
        m8 = m8 + jnp.where(merged[i] == t8, one, 0)
    cnt_eq = jnp.zeros((16,), jnp.int32)
    for j in range(NUM_EXPERTS):
        v = col(j)
        gt = v > t8
        eq = v == t8
        sel = gt | (eq & (cnt_eq < m8))
        cnt_eq = cnt_eq + jnp.where(eq, one, 0)
        zz = jnp.minimum(jnp.maximum(v, 0.0) * 2.0, 15.0)
        q = zz.astype(jnp.int32).astype(jnp.float32) * 0.5
        o = jnp.where(sel, q, jnp.float32(0.0))
        plsc.store_scatter(out_v, [rowoff + j], o)


def _sc_route(logits_flat, n_tokens):
    vals_per_worker = (n_tokens // N_WORKERS) * NUM_EXPERTS
    chunk_vals = SC_CHUNK * NUM_EXPERTS
    n_chunks = vals_per_worker // chunk_vals
    mesh = plsc.VectorSubcoreMesh(core_axis_name="c", subcore_axis_name="s")

    @functools.partial(
        pl.kernel, mesh=mesh,
        compiler_params=pltpu.CompilerParams(needs_layout_passes=False),
        out_type=jax.ShapeDtypeStruct((n_tokens * NUM_EXPERTS,), jnp.float32),
        scratch_types=[
            pltpu.VMEM((chunk_vals,), jnp.float32),
            pltpu.VMEM((chunk_vals,), jnp.float32),
        ],
    )
    def route(logits_hbm, out_hbm, in_v, out_v):
        wid = lax.axis_index("s") * 2 + lax.axis_index("c")
        base = wid * vals_per_worker

        def chunk_body(ci, carry):
            cbase = base + ci * chunk_vals
            pltpu.sync_copy(logits_hbm.at[pl.ds(cbase, chunk_vals)], in_v)

            @plsc.parallel_loop(0, SC_CHUNK // 16, unroll=2)
            def b_body(bi):
                _batch16(in_v, out_v, bi * 16)
            pltpu.sync_copy(out_v, out_hbm.at[pl.ds(cbase, chunk_vals)])
            return carry

        lax.fori_loop(0, n_chunks, chunk_body, 0)

    return route(logits_flat)


@jax.jit
def kernel(x, W, b):
    n_tokens = x.shape[0]
    logits = _tc_logits(x, W, b)
    rw = _sc_route(logits.reshape(-1), n_tokens)
    return (rw.reshape(n_tokens, NUM_EXPERTS), logits)


# speedup vs baseline: 3.3670x; 3.1505x over previous
import functools

import jax
import jax.numpy as jnp
from jax.experimental import pallas as pl
from jax.experimental.pallas import tpu as pltpu

D_MODEL = 768
NUM_EXPERTS = 64
TOP_K = 8
BLOCK_R = 4096


def _router_body(x_ref, w_ref, b_ref, logits_ref, rw_ref):
    lt = jax.lax.dot_general(
        w_ref[...], x_ref[...],
        dimension_numbers=(((0,), (1,)), ((), ())),
        preferred_element_type=jnp.float32,
    ) + b_ref[...]

    idx = jax.lax.broadcasted_iota(jnp.int32, lt.shape, 0)
    m = lt
    for _ in range(TOP_K):
        mx = jnp.max(m, axis=0, keepdims=True)
        eq = m == mx
        fi = jnp.min(jnp.where(eq, idx, NUM_EXPERTS), axis=0, keepdims=True)
        m = jnp.where(eq & (idx == fi), -jnp.inf, m)

    sel = m != lt
    q = jnp.minimum(jnp.floor(jnp.maximum(lt, 0.0) * 2.0) * 0.5, 7.5)
    rwt = jnp.where(sel, q, 0.0)
    logits_ref[...] = lt.T
    rw_ref[...] = rwt.T


@functools.partial(jax.jit, static_argnames=())
def kernel(x, W, b):
    n_tokens = x.shape[0]
    grid = (n_tokens // BLOCK_R,)
    logits, rw = pl.pallas_call(
        _router_body,
        grid=grid,
        in_specs=[
            pl.BlockSpec((BLOCK_R, D_MODEL), lambda i: (i, 0)),
            pl.BlockSpec((D_MODEL, NUM_EXPERTS), lambda i: (0, 0)),
            pl.BlockSpec((NUM_EXPERTS, 1), lambda i: (0, 0)),
        ],
        out_specs=[
            pl.BlockSpec((BLOCK_R, NUM_EXPERTS), lambda i: (i, 0)),
            pl.BlockSpec((BLOCK_R, NUM_EXPERTS), lambda i: (i, 0)),
        ],
        out_shape=[
            jax.ShapeDtypeStruct((n_tokens, NUM_EXPERTS), jnp.float32),
            jax.ShapeDtypeStruct((n_tokens, NUM_EXPERTS), jnp.float32),
        ],
        compiler_params=pltpu.CompilerParams(
            dimension_semantics=("parallel",),
        ),
    )(x, W, b.reshape(NUM_EXPERTS, 1))
    return (rw, logits)
